# baseline (speedup 1.0000x reference)
import functools

import jax
import jax.numpy as jnp
from jax import lax
from jax.experimental import pallas as pl
from jax.experimental.pallas import tpu as pltpu
from jax.experimental.pallas import tpu_sc as plsc

VOCAB = 100000
D_EMB = 128
MAX_SEQ = 2048
D_MODEL = 1024
LN_EPS = 1e-12

BATCH = 4
SEQ = 2048
N_TOK = BATCH * SEQ
NCHUNK = 2
CH_TOK = N_TOK // NCHUNK
NW = 32
TOK_PER_TILE = CH_TOK // NW

TC_BLOCK = 2048
TC_STEPS = CH_TOK // TC_BLOCK


@functools.cache
def _make_sc_gather(chunk_idx):
  mesh = plsc.VectorSubcoreMesh(core_axis_name="c", subcore_axis_name="s")

  @functools.partial(
      pl.kernel,
      mesh=mesh,
      out_type=(
          jax.ShapeDtypeStruct((CH_TOK, D_EMB), jnp.float32),
          jax.ShapeDtypeStruct((CH_TOK, D_EMB), jnp.float32),
      ),
      scratch_types=[
          pltpu.VMEM((TOK_PER_TILE,), jnp.int32),
          pltpu.VMEM((TOK_PER_TILE,), jnp.int32),
          pltpu.VMEM((TOK_PER_TILE, D_EMB), jnp.float32),
          pltpu.VMEM((TOK_PER_TILE, D_EMB), jnp.float32),
          pltpu.SemaphoreType.DMA,
      ],
  )
  def gather_kernel(ids_hbm, tids_hbm, wtab_hbm, ttab_hbm, wout_hbm,
                    tout_hbm, idx_v, tix_v, wrows_v, trows_v, sem):
    c = lax.axis_index("c")
    s = lax.axis_index("s")
    wid = s * 2 + c
    tok0 = chunk_idx * CH_TOK + wid * TOK_PER_TILE
    b = tok0 // SEQ
    off = tok0 % SEQ
    pltpu.sync_copy(ids_hbm.at[b, pl.ds(off, TOK_PER_TILE)], idx_v)
    pltpu.sync_copy(tids_hbm.at[b, pl.ds(off, TOK_PER_TILE)], tix_v)
    cpw = pltpu.async_copy(wtab_hbm.at[idx_v], wrows_v, sem)
    cpt = pltpu.async_copy(ttab_hbm.at[tix_v], trows_v, sem)
    cpw.wait()
    cpt.wait()
    base = wid * TOK_PER_TILE
    pltpu.sync_copy(wrows_v, wout_hbm.at[pl.ds(base, TOK_PER_TILE)])
    pltpu.sync_copy(trows_v, tout_hbm.at[pl.ds(base, TOK_PER_TILE)])

  return gather_kernel


def _tc_body(gath_ref, typ_ref, pos_ref, scale_ref, bias_ref,
             dk_ref, db_ref, *rest):
  out_ref = rest[-1]
  x = gath_ref[...] + typ_ref[...] + pos_ref[...]
  mean = jnp.mean(x, axis=1, keepdims=True)
  xc = x - mean
  var = jnp.mean(xc * xc, axis=1, keepdims=True)
  y = xc * lax.rsqrt(var + LN_EPS)
  y = y * scale_ref[...] + bias_ref[...]
  out_ref[...] = (
      jnp.dot(y, dk_ref[...], preferred_element_type=jnp.float32)
      + db_ref[...])


def _tc_call(chunk_idx, gathered, typ, pos2, scale2, bias2,
             dense_kernel, db2, buf):
  pos_blocks = MAX_SEQ // TC_BLOCK if TC_BLOCK < MAX_SEQ else 1
  base = chunk_idx * TC_STEPS

  in_specs = [
      pl.BlockSpec((TC_BLOCK, D_EMB), lambda i: (i, 0)),
      pl.BlockSpec((TC_BLOCK, D_EMB), lambda i: (i, 0)),
      pl.BlockSpec((TC_BLOCK, D_EMB), lambda i: (i % pos_blocks, 0)),
      pl.BlockSpec((1, D_EMB), lambda i: (0, 0)),
      pl.BlockSpec((1, D_EMB), lambda i: (0, 0)),
      pl.BlockSpec((D_EMB, D_MODEL), lambda i: (0, 0)),
      pl.BlockSpec((1, D_MODEL), lambda i: (0, 0)),
  ]
  args = [gathered, typ, pos2, scale2, bias2, dense_kernel, db2]
  aliases = {}
  if buf is not None:
    in_specs.append(pl.BlockSpec(memory_space=pl.ANY))
    args.append(buf)
    aliases = {7: 0}

  return pl.pallas_call(
      _tc_body,
      grid=(TC_STEPS,),
      in_specs=in_specs,
      out_specs=pl.BlockSpec((TC_BLOCK, D_MODEL), lambda i: (base + i, 0)),
      out_shape=jax.ShapeDtypeStruct((N_TOK, D_MODEL), jnp.float32),
      input_output_aliases=aliases,
  )(*args)


def kernel(input_ids, type_ids, word_emb, pos_emb, type_emb, ln_scale,
           ln_bias, dense_kernel, dense_bias):
  batch, seq = input_ids.shape

  ids = input_ids.astype(jnp.int32)
  tids = type_ids.astype(jnp.int32)

  gathered = [
      _make_sc_gather(i)(ids, tids, word_emb, type_emb)
      for i in range(NCHUNK)
  ]

  pos2 = pos_emb.reshape(MAX_SEQ, D_EMB)[:seq]
  scale2 = ln_scale.reshape(1, D_EMB)
  bias2 = ln_bias.reshape(1, D_EMB)
  db2 = dense_bias.reshape(1, D_MODEL)

  buf = None
  for i in range(NCHUNK):
    wrows, trows = gathered[i]
    buf = _tc_call(i, wrows, trows, pos2, scale2, bias2,
                   dense_kernel, db2, buf)

  return buf.reshape(batch, seq, D_MODEL)


# speedup vs baseline: 4.6725x; 4.6725x over previous
import functools

import jax
import jax.numpy as jnp
from jax import lax
from jax.experimental import pallas as pl
from jax.experimental.pallas import tpu as pltpu
from jax.experimental.pallas import tpu_sc as plsc

VOCAB = 100000
D_EMB = 128
MAX_SEQ = 2048
D_MODEL = 1024
LN_EPS = 1e-12

BATCH = 4
SEQ = 2048
N_TOK = BATCH * SEQ
NCHUNK = 2
CH_TOK = N_TOK // NCHUNK
NW = 32
TOK_PER_TILE = CH_TOK // NW

TC_BLOCK = 2048
TC_STEPS = CH_TOK // TC_BLOCK


@functools.cache
def _make_sc_gather(chunk_idx):
  mesh = plsc.VectorSubcoreMesh(core_axis_name="c", subcore_axis_name="s")

  @functools.partial(
      pl.kernel,
      mesh=mesh,
      out_type=jax.ShapeDtypeStruct((CH_TOK, D_EMB), jnp.float32),
      scratch_types=[
          pltpu.VMEM((TOK_PER_TILE,), jnp.int32),
          pltpu.VMEM((TOK_PER_TILE, D_EMB), jnp.float32),
          pltpu.SemaphoreType.DMA,
      ],
  )
  def gather_kernel(ids_hbm, wtab_hbm, wout_hbm, idx_v, wrows_v, sem):
    c = lax.axis_index("c")
    s = lax.axis_index("s")
    wid = s * 2 + c
    tok0 = chunk_idx * CH_TOK + wid * TOK_PER_TILE
    b = tok0 // SEQ
    off = tok0 % SEQ
    pltpu.sync_copy(ids_hbm.at[b, pl.ds(off, TOK_PER_TILE)], idx_v)
    pltpu.async_copy(wtab_hbm.at[idx_v], wrows_v, sem).wait()
    base = wid * TOK_PER_TILE
    pltpu.sync_copy(wrows_v, wout_hbm.at[pl.ds(base, TOK_PER_TILE)])

  return gather_kernel


def _tc_body(gath_ref, tid_ref, temb_ref, pos_ref, scale_ref, bias_ref,
             dk_ref, db_ref, *rest):
  out_ref = rest[-1]
  x = gath_ref[...] + pos_ref[...]
  t = tid_ref[...].astype(jnp.float32)
  te0 = temb_ref[0:1, :]
  te1 = temb_ref[1:2, :]
  x = x + te0 + t * (te1 - te0)
  mean = jnp.mean(x, axis=1, keepdims=True)
  xc = x - mean
  var = jnp.mean(xc * xc, axis=1, keepdims=True)
  y = xc * lax.rsqrt(var + LN_EPS)
  y = y * scale_ref[...] + bias_ref[...]
  out_ref[...] = (
      jnp.dot(y, dk_ref[...], preferred_element_type=jnp.float32)
      + db_ref[...])


def _tc_call(chunk_idx, gathered, tids, temb, pos2, scale2, bias2,
             dense_kernel, db2, buf):
  pos_blocks = MAX_SEQ // TC_BLOCK if TC_BLOCK < MAX_SEQ else 1
  base = chunk_idx * TC_STEPS

  in_specs = [
      pl.BlockSpec((TC_BLOCK, D_EMB), lambda i: (i, 0)),
      pl.BlockSpec((TC_BLOCK, 1), lambda i: (base + i, 0)),
      pl.BlockSpec((2, D_EMB), lambda i: (0, 0)),
      pl.BlockSpec((TC_BLOCK, D_EMB), lambda i: (i % pos_blocks, 0)),
      pl.BlockSpec((1, D_EMB), lambda i: (0, 0)),
      pl.BlockSpec((1, D_EMB), lambda i: (0, 0)),
      pl.BlockSpec((D_EMB, D_MODEL), lambda i: (0, 0)),
      pl.BlockSpec((1, D_MODEL), lambda i: (0, 0)),
  ]
  args = [gathered, tids, temb, pos2, scale2, bias2, dense_kernel, db2]
  aliases = {}
  if buf is not None:
    in_specs.append(pl.BlockSpec(memory_space=pl.ANY))
    args.append(buf)
    aliases = {8: 0}

  return pl.pallas_call(
      _tc_body,
      grid=(TC_STEPS,),
      in_specs=in_specs,
      out_specs=pl.BlockSpec((TC_BLOCK, D_MODEL), lambda i: (base + i, 0)),
      out_shape=jax.ShapeDtypeStruct((N_TOK, D_MODEL), jnp.float32),
      input_output_aliases=aliases,
  )(*args)


def kernel(input_ids, type_ids, word_emb, pos_emb, type_emb, ln_scale,
           ln_bias, dense_kernel, dense_bias):
  batch, seq = input_ids.shape

  ids = input_ids.astype(jnp.int32)
  tids = type_ids.reshape(batch * seq, 1).astype(jnp.int32)

  gathered = [
      _make_sc_gather(i)(ids, word_emb)
      for i in range(NCHUNK)
  ]

  pos2 = pos_emb.reshape(MAX_SEQ, D_EMB)[:seq]
  scale2 = ln_scale.reshape(1, D_EMB)
  bias2 = ln_bias.reshape(1, D_EMB)
  db2 = dense_bias.reshape(1, D_MODEL)

  buf = None
  for i in range(NCHUNK):
    buf = _tc_call(i, gathered[i], tids, type_emb, pos2, scale2, bias2,
                   dense_kernel, db2, buf)

  return buf.reshape(batch, seq, D_MODEL)
